# Initial kernel scaffold; baseline (speedup 1.0000x reference)
import jax
import jax.numpy as jnp
from jax.experimental import pallas as pl


def kernel(x, W):
    raise NotImplementedError("write your pallas kernel here")


# speedup vs baseline: 7.2540x; 7.2540x over previous
import functools

import jax
import jax.numpy as jnp
from jax import lax
from jax.experimental import pallas as pl
from jax.experimental.pallas import tpu as pltpu
from jax.experimental.pallas import tpu_sc as plsc

N = 320000
NT = 17
R = 8
D = 128
L = 16
NC, NS = 2, 16
NW = NC * NS
ROWS_PER_W = N // NW
C = 200
NCHUNK = ROWS_PER_W // C

NPAIR = 8
TBL_ROWS = NPAIR * 64 + R


def _sc_body(w_hbm, x_hbm, out_hbm, wbuf, tbl, xbuf, outbuf):
    wid = lax.axis_index("s") * NC + lax.axis_index("c")
    base = wid * ROWS_PER_W

    pltpu.sync_copy(w_hbm, wbuf)

    def build_pair(p, _):
        def build_ab(ab, _):
            a = ab // R
            b = ab - a * R
            src_a = ((2 * p) * R + a) * D
            src_b = ((2 * p + 1) * R + b) * D
            dst = (p * 64 + ab) * D
            for d in range(D // L):
                va = wbuf[pl.ds(src_a + d * L, L)]
                vb = wbuf[pl.ds(src_b + d * L, L)]
                tbl[pl.ds(dst + d * L, L)] = va + vb
            return 0
        lax.fori_loop(0, 64, build_ab, 0)
        return 0
    lax.fori_loop(0, NPAIR, build_pair, 0)

    def build_single(r, _):
        for d in range(D // L):
            tbl[pl.ds((NPAIR * 64 + r) * D + d * L, L)] = \
                wbuf[pl.ds((16 * R + r) * D + d * L, L)]
        return 0
    lax.fori_loop(0, R, build_single, 0)

    def chunk_body(j, _):
        row0 = base + j * C
        pltpu.sync_copy(x_hbm.at[pl.ds(row0 * NT, C * NT)],
                        xbuf.at[pl.ds(0, C * NT)])

        def row_body(r, _):
            xoff = r * NT
            xv0 = xbuf[pl.ds(xoff, L)]
            xv1 = xbuf[pl.ds(xoff + L, L)]
            offs = []
            for p in range(NPAIR):
                a = xv0[2 * p]
                b = xv0[2 * p + 1]
                offs.append((p * 64) * D + (a * R + b) * D)
            offs.append((NPAIR * 64) * D + xv1[0] * D)
            for d in range(D // L):
                acc = tbl[pl.ds(offs[0] + d * L, L)]
                for t in range(1, NPAIR + 1):
                    acc = acc + tbl[pl.ds(offs[t] + d * L, L)]
                outbuf[pl.ds(r * D + d * L, L)] = acc
            return 0
        lax.fori_loop(0, C, row_body, 0)

        pltpu.sync_copy(outbuf, out_hbm.at[pl.ds(row0 * D, C * D)])
        return 0
    lax.fori_loop(0, NCHUNK, chunk_body, 0)


@jax.jit
def _encode(x_flat, w_flat):
    mesh = plsc.VectorSubcoreMesh(
        core_axis_name="c", subcore_axis_name="s", num_cores=NC, num_subcores=NS)
    f = pl.kernel(
        _sc_body,
        out_type=jax.ShapeDtypeStruct((N * D,), jnp.float32),
        mesh=mesh,
        scratch_types=[
            pltpu.VMEM((NT * R * D,), jnp.float32),
            pltpu.VMEM((TBL_ROWS * D,), jnp.float32),
            pltpu.VMEM((C * NT + L,), jnp.int32),
            pltpu.VMEM((C * D,), jnp.float32),
        ],
    )
    return f(w_flat, x_flat)


def kernel(x, W):
    x_flat = x.reshape(-1).astype(jnp.int32)
    w_flat = W.reshape(-1)
    out = _encode(x_flat, w_flat)
    return out.reshape(N, D)
